# baseline (speedup 1.0000x reference)
import jax
import jax.numpy as jnp
from jax import lax
from jax.experimental import pallas as pl
from jax.experimental.pallas import tpu as pltpu
from jax.experimental.pallas import tpu_sc as plsc

DIM = 64
NC, NS, L = 2, 16, 16
NW = NC * NS
T = 32768
PER_W = T // NW
CHUNK = 128
N_CHUNKS = PER_W // CHUNK
GROUPS = CHUNK // L
LAMBDA = 1e-4

ENT_ROWS = 1000000
GATHER_ROWS = 100000
REL_ROWS = 100000

_SC_PARAMS = pltpu.CompilerParams(needs_layout_passes=False)
_MESH = dict(core_axis_name="c", subcore_axis_name="s")


def _tc_ent_norm_body(e_b, out_ref, acc):
    g = pl.program_id(0)

    @pl.when(g == 0)
    def _():
        acc[0] = 0.0

    acc[0] += jnp.sum(e_b[...] * e_b[...])

    @pl.when(g == 7)
    def _():
        out_ref[...] = jnp.full((1, 1), acc[0], jnp.float32)


def _tc_ent_norm(table_t):
    return pl.pallas_call(
        _tc_ent_norm_body,
        grid=(8,),
        in_specs=[pl.BlockSpec((8, ENT_ROWS), lambda g: (g, 0))],
        out_specs=pl.BlockSpec((1, 1), lambda g: (0, 0)),
        out_shape=jax.ShapeDtypeStruct((1, 1), jnp.float32),
        scratch_shapes=[pltpu.SMEM((8,), jnp.float32)],
        compiler_params=pltpu.CompilerParams(
            vmem_limit_bytes=100 * 1024 * 1024),
    )(table_t)


CAT_ROWS = 102400
CAT_BLK = 6400


def _tc_cat_body(ea_b, eb_b, ra_b, rb_b, ce_ref, cr_ref):
    ce_ref[...] = jnp.transpose(
        jnp.concatenate([ea_b[...], eb_b[...]], axis=0))
    cr_ref[...] = jnp.transpose(
        jnp.concatenate([ra_b[...], rb_b[...]], axis=0))


def _tc_cat(ent_re_t, ent_im_t, rel_re_t, rel_im_t):
    return pl.pallas_call(
        _tc_cat_body,
        grid=(CAT_ROWS // CAT_BLK,),
        in_specs=[
            pl.BlockSpec((DIM, CAT_BLK), lambda g: (0, g)),
            pl.BlockSpec((DIM, CAT_BLK), lambda g: (0, g)),
            pl.BlockSpec((DIM, CAT_BLK), lambda g: (0, g)),
            pl.BlockSpec((DIM, CAT_BLK), lambda g: (0, g)),
        ],
        out_specs=(
            pl.BlockSpec((CAT_BLK, 2 * DIM), lambda g: (g, 0)),
            pl.BlockSpec((CAT_BLK, 2 * DIM), lambda g: (g, 0)),
        ),
        out_shape=(
            jax.ShapeDtypeStruct((CAT_ROWS, 2 * DIM), jnp.float32),
            jax.ShapeDtypeStruct((CAT_ROWS, 2 * DIM), jnp.float32),
        ),
        compiler_params=pltpu.CompilerParams(
            vmem_limit_bytes=100 * 1024 * 1024),
    )(ent_re_t, ent_im_t, rel_re_t, rel_im_t)


REL_CH = 200
REL_NCHUNK = REL_ROWS // REL_CH


def _sc_gather_body(h_hbm, r_hbm, t_hbm, cat_ent, cat_rel,
                    out_hbm, parts_hbm,
                    idx_h, idx_r, idx_t, hb, rb, tb, sc_v, nb, acc_v, sem):
    w = lax.axis_index("s") * NC + lax.axis_index("c")
    base = w * PER_W
    sign = jnp.where(base < T // 2, 1.0, -1.0).astype(jnp.float32)
    lane = lax.iota(jnp.int32, L)

    def start_chunk(c, p):
        off = base + c * CHUNK
        pltpu.sync_copy(h_hbm.at[pl.ds(off, CHUNK)], idx_h.at[p])
        pltpu.sync_copy(r_hbm.at[pl.ds(off, CHUNK)], idx_r.at[p])
        pltpu.sync_copy(t_hbm.at[pl.ds(off, CHUNK)], idx_t.at[p])
        return [
            pltpu.async_copy(cat_ent.at[idx_h.at[p]], hb.at[p], sem),
            pltpu.async_copy(cat_rel.at[idx_r.at[p]], rb.at[p], sem),
            pltpu.async_copy(cat_ent.at[idx_t.at[p]], tb.at[p], sem),
        ]

    cps = start_chunk(0, 0)
    for c in range(N_CHUNKS):
        p = c % 2
        if c + 1 < N_CHUNKS:
            nxt = start_chunk(c + 1, (c + 1) % 2)
        for cp in cps:
            cp.wait()

        def g_body(g, carry2, c=c, p=p):
            def j_body(j, svec):
                i = g * L + j
                acc = jnp.zeros((L,), jnp.float32)
                for k in range(DIM // L):
                    re_sl = pl.ds(k * L, L)
                    im_sl = pl.ds(DIM + k * L, L)
                    a = hb[p, i, re_sl]
                    b = hb[p, i, im_sl]
                    cr = rb[p, i, re_sl]
                    ci = rb[p, i, im_sl]
                    e = tb[p, i, re_sl]
                    f = tb[p, i, im_sl]
                    acc = acc + cr * (a * e + b * f) + ci * (a * f - b * e)
                s = jnp.sum(acc)
                return svec + jnp.where(lane == j, s, 0.0)

            svec = lax.fori_loop(0, L, j_body, jnp.zeros((L,), jnp.float32))
            sc_v[pl.ds(c * CHUNK + g * L, L)] = svec * sign
            return carry2

        lax.fori_loop(0, GROUPS, g_body, 0)
        if c + 1 < N_CHUNKS:
            cps = nxt
    pltpu.sync_copy(sc_v, out_hbm.at[pl.ds(base, PER_W)])

    def rel_body(k, accs):
        c = w + k * NW

        def do(accs):
            a_re, a_im = accs
            pltpu.sync_copy(cat_rel.at[pl.ds(c * REL_CH, REL_CH)], nb)

            def row_body(j, accs):
                a_re, a_im = accs
                for q in range(DIM // L):
                    vr = nb[j, pl.ds(q * L, L)]
                    vi = nb[j, pl.ds(DIM + q * L, L)]
                    a_re = a_re + vr * vr
                    a_im = a_im + vi * vi
                return (a_re, a_im)

            return lax.fori_loop(0, REL_CH, row_body, (a_re, a_im))

        return lax.cond(c < REL_NCHUNK, do, lambda a: a, accs)

    z = jnp.zeros((L,), jnp.float32)
    a_re, a_im = lax.fori_loop(0, (REL_NCHUNK + NW - 1) // NW, rel_body, (z, z))
    acc_v[pl.ds(0, L)] = a_re
    acc_v[pl.ds(L, L)] = a_im
    pltpu.sync_copy(acc_v, parts_hbm.at[w])


def _sc_gather(h_idx, r_idx, t_idx, cat_ent, cat_rel):
    kfn = pl.kernel(
        _sc_gather_body,
        out_type=(
            jax.ShapeDtypeStruct((T,), jnp.float32),
            jax.ShapeDtypeStruct((NW, 2 * L), jnp.float32),
        ),
        mesh=plsc.VectorSubcoreMesh(**_MESH),
        scratch_types=[
            pltpu.VMEM((2, CHUNK), jnp.int32),
            pltpu.VMEM((2, CHUNK), jnp.int32),
            pltpu.VMEM((2, CHUNK), jnp.int32),
            pltpu.VMEM((2, CHUNK, 2 * DIM), jnp.float32),
            pltpu.VMEM((2, CHUNK, 2 * DIM), jnp.float32),
            pltpu.VMEM((2, CHUNK, 2 * DIM), jnp.float32),
            pltpu.VMEM((PER_W,), jnp.float32),
            pltpu.VMEM((REL_CH, 2 * DIM), jnp.float32),
            pltpu.VMEM((2 * L,), jnp.float32),
            pltpu.SemaphoreType.DMA,
        ],
        compiler_params=_SC_PARAMS,
    )
    return kfn(h_idx, r_idx, t_idx, cat_ent, cat_rel)


def _tc_final_body(sc_b, parts_b, pe_b, pi_b, out_ref):
    loss_sum = jnp.sum(jnp.log(jnp.exp(-sc_b[...]) + 1.0))
    p = parts_b[...]
    ss_rr = jnp.sum(p[:, 0:L])
    ss_ri = jnp.sum(p[:, L:2 * L])
    loss = loss_sum / T + LAMBDA * (
        jnp.sqrt(pe_b[0, 0]) + jnp.sqrt(pi_b[0, 0])
        + jnp.sqrt(ss_rr) + jnp.sqrt(ss_ri))
    out_ref[...] = jnp.full((1, 1), loss, jnp.float32)


def _tc_final(scores2d, parts, pe, pi):
    return pl.pallas_call(
        _tc_final_body,
        out_shape=jax.ShapeDtypeStruct((1, 1), jnp.float32),
    )(scores2d, parts, pe, pi)


def kernel(positive_triples, negative_triples, ent_re, ent_im, rel_re, rel_im):
    pos_t = positive_triples.T
    neg_t = negative_triples.T
    h_idx = jnp.concatenate([pos_t[0], neg_t[0]])
    r_idx = jnp.concatenate([pos_t[1], neg_t[1]])
    t_idx = jnp.concatenate([pos_t[2], neg_t[2]])
    cat_ent, cat_rel = _tc_cat(ent_re.T, ent_im.T, rel_re.T, rel_im.T)
    scores, parts = _sc_gather(h_idx, r_idx, t_idx, cat_ent, cat_rel)
    pe = _tc_ent_norm(ent_re.T)
    pi = _tc_ent_norm(ent_im.T)
    out = _tc_final(scores.reshape(T // 128, 128), parts, pe, pi)
    return out[0, 0]


# speedup vs baseline: 1.0138x; 1.0138x over previous
import jax
import jax.numpy as jnp
from jax import lax
from jax.experimental import pallas as pl
from jax.experimental.pallas import tpu as pltpu
from jax.experimental.pallas import tpu_sc as plsc

DIM = 64
NC, NS, L = 2, 16, 16
NW = NC * NS
T = 32768
PER_W = T // NW
CHUNK = 128
N_CHUNKS = PER_W // CHUNK
GROUPS = CHUNK // L
LAMBDA = 1e-4

ENT_ROWS = 1000000
GATHER_ROWS = 100000
REL_ROWS = 100000

_SC_PARAMS = pltpu.CompilerParams(needs_layout_passes=False)
_MESH = dict(core_axis_name="c", subcore_axis_name="s")


def _tc_ent_norm_body(e_b, out_ref, acc):
    g = pl.program_id(0)

    @pl.when(g == 0)
    def _():
        acc[0] = 0.0

    acc[0] += jnp.sum(e_b[...] * e_b[...])

    @pl.when(g == 7)
    def _():
        out_ref[...] = jnp.full((1, 1), acc[0], jnp.float32)


def _tc_ent_norm(table_t):
    return pl.pallas_call(
        _tc_ent_norm_body,
        grid=(8,),
        in_specs=[pl.BlockSpec((8, ENT_ROWS), lambda g: (g, 0))],
        out_specs=pl.BlockSpec((1, 1), lambda g: (0, 0)),
        out_shape=jax.ShapeDtypeStruct((1, 1), jnp.float32),
        scratch_shapes=[pltpu.SMEM((8,), jnp.float32)],
        compiler_params=pltpu.CompilerParams(
            vmem_limit_bytes=100 * 1024 * 1024),
    )(table_t)


CAT_ROWS = 102400
CAT_BLK = 6400


def _tc_cat_body(ea_b, eb_b, ra_b, rb_b, ce_ref, cr_ref):
    ce_ref[...] = jnp.transpose(
        jnp.concatenate([ea_b[...], eb_b[...]], axis=0))
    cr_ref[...] = jnp.transpose(
        jnp.concatenate([ra_b[...], rb_b[...]], axis=0))


def _tc_cat(ent_re_t, ent_im_t, rel_re_t, rel_im_t):
    return pl.pallas_call(
        _tc_cat_body,
        grid=(CAT_ROWS // CAT_BLK,),
        in_specs=[
            pl.BlockSpec((DIM, CAT_BLK), lambda g: (0, g)),
            pl.BlockSpec((DIM, CAT_BLK), lambda g: (0, g)),
            pl.BlockSpec((DIM, CAT_BLK), lambda g: (0, g)),
            pl.BlockSpec((DIM, CAT_BLK), lambda g: (0, g)),
        ],
        out_specs=(
            pl.BlockSpec((CAT_BLK, 2 * DIM), lambda g: (g, 0)),
            pl.BlockSpec((CAT_BLK, 2 * DIM), lambda g: (g, 0)),
        ),
        out_shape=(
            jax.ShapeDtypeStruct((CAT_ROWS, 2 * DIM), jnp.float32),
            jax.ShapeDtypeStruct((CAT_ROWS, 2 * DIM), jnp.float32),
        ),
        compiler_params=pltpu.CompilerParams(
            vmem_limit_bytes=100 * 1024 * 1024),
    )(ent_re_t, ent_im_t, rel_re_t, rel_im_t)


REL_CH = 200
REL_NCHUNK = REL_ROWS // REL_CH


def _sc_gather_body(h_hbm, r_hbm, t_hbm, cat_ent, cat_rel,
                    out_hbm, parts_hbm,
                    idx_h, idx_r, idx_t, hb, rb, tb, sc_v, nb, acc_v, sem):
    w = lax.axis_index("s") * NC + lax.axis_index("c")
    base = w * PER_W
    sign = jnp.where(base < T // 2, 1.0, -1.0).astype(jnp.float32)
    lane = lax.iota(jnp.int32, L)

    def chunk_body(c, carry):
        off = base + c * CHUNK
        pltpu.sync_copy(h_hbm.at[pl.ds(off, CHUNK)], idx_h)
        pltpu.sync_copy(r_hbm.at[pl.ds(off, CHUNK)], idx_r)
        pltpu.sync_copy(t_hbm.at[pl.ds(off, CHUNK)], idx_t)
        cps = [
            pltpu.async_copy(cat_ent.at[idx_h], hb, sem),
            pltpu.async_copy(cat_rel.at[idx_r], rb, sem),
            pltpu.async_copy(cat_ent.at[idx_t], tb, sem),
        ]
        for cp in cps:
            cp.wait()

        def g_body(g, carry2):
            def j_body(j, svec):
                i = g * L + j
                acc = jnp.zeros((L,), jnp.float32)
                for k in range(DIM // L):
                    re_sl = pl.ds(k * L, L)
                    im_sl = pl.ds(DIM + k * L, L)
                    a = hb[i, re_sl]
                    b = hb[i, im_sl]
                    cr = rb[i, re_sl]
                    ci = rb[i, im_sl]
                    e = tb[i, re_sl]
                    f = tb[i, im_sl]
                    acc = acc + cr * (a * e + b * f) + ci * (a * f - b * e)
                s = jnp.sum(acc)
                return svec + jnp.where(lane == j, s, 0.0)

            svec = lax.fori_loop(0, L, j_body, jnp.zeros((L,), jnp.float32))
            sc_v[pl.ds(c * CHUNK + g * L, L)] = svec * sign
            return carry2

        lax.fori_loop(0, GROUPS, g_body, 0)
        return carry

    lax.fori_loop(0, N_CHUNKS, chunk_body, 0)
    pltpu.sync_copy(sc_v, out_hbm.at[pl.ds(base, PER_W)])

    def rel_body(k, accs):
        c = w + k * NW

        def do(accs):
            a_re, a_im = accs
            pltpu.sync_copy(cat_rel.at[pl.ds(c * REL_CH, REL_CH)], nb)

            def row_body(j, accs):
                a_re, a_im = accs
                for q in range(DIM // L):
                    vr = nb[j, pl.ds(q * L, L)]
                    vi = nb[j, pl.ds(DIM + q * L, L)]
                    a_re = a_re + vr * vr
                    a_im = a_im + vi * vi
                return (a_re, a_im)

            return lax.fori_loop(0, REL_CH, row_body, (a_re, a_im))

        return lax.cond(c < REL_NCHUNK, do, lambda a: a, accs)

    z = jnp.zeros((L,), jnp.float32)
    a_re, a_im = lax.fori_loop(0, (REL_NCHUNK + NW - 1) // NW, rel_body, (z, z))
    acc_v[pl.ds(0, L)] = a_re
    acc_v[pl.ds(L, L)] = a_im
    pltpu.sync_copy(acc_v, parts_hbm.at[w])


def _sc_gather(h_idx, r_idx, t_idx, cat_ent, cat_rel):
    kfn = pl.kernel(
        _sc_gather_body,
        out_type=(
            jax.ShapeDtypeStruct((T,), jnp.float32),
            jax.ShapeDtypeStruct((NW, 2 * L), jnp.float32),
        ),
        mesh=plsc.VectorSubcoreMesh(**_MESH),
        scratch_types=[
            pltpu.VMEM((CHUNK,), jnp.int32),
            pltpu.VMEM((CHUNK,), jnp.int32),
            pltpu.VMEM((CHUNK,), jnp.int32),
            pltpu.VMEM((CHUNK, 2 * DIM), jnp.float32),
            pltpu.VMEM((CHUNK, 2 * DIM), jnp.float32),
            pltpu.VMEM((CHUNK, 2 * DIM), jnp.float32),
            pltpu.VMEM((PER_W,), jnp.float32),
            pltpu.VMEM((REL_CH, 2 * DIM), jnp.float32),
            pltpu.VMEM((2 * L,), jnp.float32),
            pltpu.SemaphoreType.DMA,
        ],
        compiler_params=_SC_PARAMS,
    )
    return kfn(h_idx, r_idx, t_idx, cat_ent, cat_rel)


def _tc_final_body(sc_b, parts_b, pe_b, pi_b, out_ref):
    loss_sum = jnp.sum(jnp.log(jnp.exp(-sc_b[...]) + 1.0))
    p = parts_b[...]
    ss_rr = jnp.sum(p[:, 0:L])
    ss_ri = jnp.sum(p[:, L:2 * L])
    loss = loss_sum / T + LAMBDA * (
        jnp.sqrt(pe_b[0, 0]) + jnp.sqrt(pi_b[0, 0])
        + jnp.sqrt(ss_rr) + jnp.sqrt(ss_ri))
    out_ref[...] = jnp.full((1, 1), loss, jnp.float32)


def _tc_final(scores2d, parts, pe, pi):
    return pl.pallas_call(
        _tc_final_body,
        out_shape=jax.ShapeDtypeStruct((1, 1), jnp.float32),
    )(scores2d, parts, pe, pi)


def kernel(positive_triples, negative_triples, ent_re, ent_im, rel_re, rel_im):
    pos_t = positive_triples.T
    neg_t = negative_triples.T
    h_idx = jnp.concatenate([pos_t[0], neg_t[0]])
    r_idx = jnp.concatenate([pos_t[1], neg_t[1]])
    t_idx = jnp.concatenate([pos_t[2], neg_t[2]])
    cat_ent, cat_rel = _tc_cat(ent_re.T, ent_im.T, rel_re.T, rel_im.T)
    scores, parts = _sc_gather(h_idx, r_idx, t_idx, cat_ent, cat_rel)
    pe = _tc_ent_norm(ent_re.T)
    pi = _tc_ent_norm(ent_im.T)
    out = _tc_final(scores.reshape(T // 128, 128), parts, pe, pi)
    return out[0, 0]
